# baseline (speedup 1.0000x reference)
import functools
import jax
import jax.numpy as jnp
from jax import lax
from jax.experimental import pallas as pl
from jax.experimental.pallas import tpu as pltpu
from jax.experimental.pallas import tpu_sc as plsc

N = 1048576
NBINS = 10
TAIL = 5.0
MINW = 0.001
MINH = 0.001
MIND = 0.001
NWORKERS = 32
WELEMS = N // NWORKERS
CHUNK = 2048
NCHUNKS = WELEMS // CHUNK
NGROUPS = CHUNK // 16

_WSCALE = (1.0 - MINW * NBINS) * (2.0 * TAIL)
_HSCALE = (1.0 - MINH * NBINS) * (2.0 * TAIL)


def _tree_reduce(vals, op):
    vals = list(vals)
    while len(vals) > 1:
        nxt = [op(vals[i], vals[i + 1]) for i in range(0, len(vals) - 1, 2)]
        if len(vals) % 2:
            nxt.append(vals[-1])
        vals = nxt
    return vals[0]


def _log1p_small(v):
    s = v / (2.0 + v)
    s2 = s * s
    return 2.0 * s * (1.0 + s2 * (1.0 / 3.0 + s2 * (1.0 / 5.0 + s2 * (1.0 / 7.0))))


def _softplus(u):
    t = jnp.exp(-jnp.abs(u))
    return jnp.maximum(u, 0.0) + _log1p_small(t)


def _sqrt_nn(v):
    i = lax.bitcast_convert_type(v, jnp.int32)
    i = 0x5F3759DF - lax.shift_right_logical(i, 1)
    r = lax.bitcast_convert_type(i, jnp.float32)
    r = r * (1.5 - 0.5 * v * r * r)
    r = r * (1.5 - 0.5 * v * r * r)
    r = r * (1.5 - 0.5 * v * r * r)
    return v * r


def _spline_group(x, uwk, uhk, udb, lane_e):
    f32 = jnp.float32

    m = _tree_reduce(uwk, jnp.maximum)
    ew = [jnp.exp(uwk[k] - m) for k in range(NBINS)]
    sw = _tree_reduce(ew, jnp.add)
    rw = _WSCALE / sw
    cw_raw = [ew[0]]
    for k in range(1, NBINS):
        cw_raw.append(cw_raw[k - 1] + ew[k])
    cwA = [None] * (NBINS + 1)
    for k in range(1, NBINS):
        cwA[k] = (MINW * 2.0 * TAIL * k - TAIL) + cw_raw[k - 1] * rw

    mh = _tree_reduce(uhk, jnp.maximum)
    eh = [jnp.exp(uhk[k] - mh) for k in range(NBINS)]
    sh = _tree_reduce(eh, jnp.add)
    rh = _HSCALE / sh
    ch_raw = [eh[0]]
    for k in range(1, NBINS):
        ch_raw.append(ch_raw[k - 1] + eh[k])
    chA = [None] * (NBINS + 1)
    for k in range(1, NBINS):
        chA[k] = (MINH * 2.0 * TAIL * k - TAIL) + ch_raw[k - 1] * rh

    one_i = jnp.full((16,), 1, jnp.int32)
    zero_i = jnp.full((16,), 0, jnp.int32)
    idx = jnp.where(x >= chA[1], one_i, zero_i)
    for k in range(2, NBINS):
        idx = idx + jnp.where(x >= chA[k], one_i, zero_i)

    mks = [idx == k for k in range(NBINS)]

    neg_t = jnp.full((16,), -TAIL, f32)
    pos_t = jnp.full((16,), TAIL, f32)
    ch_lo = neg_t
    cw_lo = neg_t
    for k in range(1, NBINS):
        ch_lo = jnp.where(mks[k], chA[k], ch_lo)
        cw_lo = jnp.where(mks[k], cwA[k], cw_lo)
    ch_hi = pos_t
    cw_hi = pos_t
    for k in range(1, NBINS):
        ch_hi = jnp.where(mks[k - 1], chA[k], ch_hi)
        cw_hi = jnp.where(mks[k - 1], cwA[k], cw_hi)

    heights = ch_hi - ch_lo
    widths = cw_hi - cw_lo

    d_lo_raw = plsc.load_gather(udb, [lane_e, idx])
    d_hi_raw = plsc.load_gather(udb, [lane_e, idx + 1])
    one_f = jnp.full((16,), 1.0, f32)
    d_lo = jnp.where(mks[0], one_f, MIND + _softplus(d_lo_raw))
    d_hi = jnp.where(mks[NBINS - 1], one_f, MIND + _softplus(d_hi_raw))

    delta = heights / widths
    dx = x - ch_lo
    two = d_lo + d_hi - 2.0 * delta
    aq = dx * two + heights * (delta - d_lo)
    bq = heights * d_lo - dx * two
    cq = -delta * dx
    disc = bq * bq - 4.0 * aq * cq
    root = (2.0 * cq) / (-bq - _sqrt_nn(jnp.maximum(disc, 0.0)))
    return root * widths + cw_lo


def _tec_kernel(x_hbm, uw_hbm, uh_hbm, ud_hbm, out_hbm, xb, uwb, uhb, udb, ob):
    c = lax.axis_index("c")
    s = lax.axis_index("s")
    wid = s * 2 + c
    wbase = wid * WELEMS
    lane = lax.iota(jnp.int32, 16)

    def chunk_body(ci, carry):
        ebase = wbase + ci * CHUNK
        pltpu.sync_copy(x_hbm.at[0, 0, pl.ds(ebase, CHUNK)], xb)
        pltpu.sync_copy(uw_hbm.at[0, 0, pl.ds(ebase, CHUNK), :], uwb)
        pltpu.sync_copy(uh_hbm.at[0, 0, pl.ds(ebase, CHUNK), :], uhb)
        pltpu.sync_copy(ud_hbm.at[0, 0, pl.ds(ebase, CHUNK), :], udb)

        def group_body(g, carry2):
            b = g * 16
            lane_e = b + lane
            kcol = [jnp.full((16,), k, jnp.int32) for k in range(NBINS)]
            uwk = [plsc.load_gather(uwb, [lane_e, kcol[k]]) for k in range(NBINS)]
            uhk = [plsc.load_gather(uhb, [lane_e, kcol[k]]) for k in range(NBINS)]
            x = xb[pl.ds(b, 16)]
            out = _spline_group(x, uwk, uhk, udb, lane_e)
            ob[pl.ds(b, 16)] = out
            return carry2

        lax.fori_loop(0, NGROUPS, group_body, 0, unroll=2)
        pltpu.sync_copy(ob, out_hbm.at[0, 0, pl.ds(ebase, CHUNK)])
        return carry

    lax.fori_loop(0, NCHUNKS, chunk_body, 0)


@jax.jit
def _run(x, uwf, uhf, udf):
    mesh = plsc.VectorSubcoreMesh(core_axis_name="c", subcore_axis_name="s")
    grid_kernel = pl.kernel(
        _tec_kernel,
        out_type=jax.ShapeDtypeStruct((1, 1, N), jnp.float32),
        mesh=mesh,
        compiler_params=pltpu.CompilerParams(
            needs_layout_passes=False, use_tc_tiling_on_sc=False),
        scratch_types=[
            pltpu.VMEM((CHUNK,), jnp.float32),
            pltpu.VMEM((CHUNK, NBINS), jnp.float32),
            pltpu.VMEM((CHUNK, NBINS), jnp.float32),
            pltpu.VMEM((CHUNK, NBINS + 1), jnp.float32),
            pltpu.VMEM((CHUNK,), jnp.float32),
        ],
    )
    return grid_kernel(x, uwf, uhf, udf)


def kernel(inputs, unnormalized_widths, unnormalized_heights, unnormalized_derivatives):
    return _run(inputs, unnormalized_widths, unnormalized_heights,
                unnormalized_derivatives)


# speedup vs baseline: 6.5482x; 6.5482x over previous
import functools
import jax
import jax.numpy as jnp
from jax import lax
from jax.experimental import pallas as pl
from jax.experimental.pallas import tpu as pltpu
from jax.experimental.pallas import tpu_sc as plsc

N = 1048576
NBINS = 10
TAIL = 5.0
MINW = 0.001
MINH = 0.001
MIND = 0.001
NWORKERS = 32
WELEMS = N // NWORKERS
CHUNK = 2048
NCHUNKS = WELEMS // CHUNK
NGROUPS = CHUNK // 16

_WSCALE = (1.0 - MINW * NBINS) * (2.0 * TAIL)
_HSCALE = (1.0 - MINH * NBINS) * (2.0 * TAIL)


def _tree_reduce(vals, op):
    vals = list(vals)
    while len(vals) > 1:
        nxt = [op(vals[i], vals[i + 1]) for i in range(0, len(vals) - 1, 2)]
        if len(vals) % 2:
            nxt.append(vals[-1])
        vals = nxt
    return vals[0]


def _log1p_small(v):
    s = v / (2.0 + v)
    s2 = s * s
    return 2.0 * s * (1.0 + s2 * (1.0 / 3.0 + s2 * (1.0 / 5.0 + s2 * (1.0 / 7.0))))


def _softplus(u):
    t = jnp.exp(-jnp.abs(u))
    return jnp.maximum(u, 0.0) + _log1p_small(t)


def _sqrt_nn(v):
    i = lax.bitcast_convert_type(v, jnp.int32)
    i = 0x5F3759DF - lax.shift_right_logical(i, 1)
    r = lax.bitcast_convert_type(i, jnp.float32)
    r = r * (1.5 - 0.5 * v * r * r)
    r = r * (1.5 - 0.5 * v * r * r)
    r = r * (1.5 - 0.5 * v * r * r)
    return v * r


def _spline_group(x, uwk, uhk, udb, lane_e):
    f32 = jnp.float32

    m = _tree_reduce(uwk, jnp.maximum)
    ew = [jnp.exp(uwk[k] - m) for k in range(NBINS)]
    sw = _tree_reduce(ew, jnp.add)
    rw = _WSCALE / sw
    cw_raw = [ew[0]]
    for k in range(1, NBINS):
        cw_raw.append(cw_raw[k - 1] + ew[k])
    cwA = [None] * (NBINS + 1)
    for k in range(1, NBINS):
        cwA[k] = (MINW * 2.0 * TAIL * k - TAIL) + cw_raw[k - 1] * rw

    mh = _tree_reduce(uhk, jnp.maximum)
    eh = [jnp.exp(uhk[k] - mh) for k in range(NBINS)]
    sh = _tree_reduce(eh, jnp.add)
    rh = _HSCALE / sh
    ch_raw = [eh[0]]
    for k in range(1, NBINS):
        ch_raw.append(ch_raw[k - 1] + eh[k])
    chA = [None] * (NBINS + 1)
    for k in range(1, NBINS):
        chA[k] = (MINH * 2.0 * TAIL * k - TAIL) + ch_raw[k - 1] * rh

    one_i = jnp.full((16,), 1, jnp.int32)
    zero_i = jnp.full((16,), 0, jnp.int32)
    idx = jnp.where(x >= chA[1], one_i, zero_i)
    for k in range(2, NBINS):
        idx = idx + jnp.where(x >= chA[k], one_i, zero_i)

    mks = [idx == k for k in range(NBINS)]

    neg_t = jnp.full((16,), -TAIL, f32)
    pos_t = jnp.full((16,), TAIL, f32)
    ch_lo = neg_t
    cw_lo = neg_t
    for k in range(1, NBINS):
        ch_lo = jnp.where(mks[k], chA[k], ch_lo)
        cw_lo = jnp.where(mks[k], cwA[k], cw_lo)
    ch_hi = pos_t
    cw_hi = pos_t
    for k in range(1, NBINS):
        ch_hi = jnp.where(mks[k - 1], chA[k], ch_hi)
        cw_hi = jnp.where(mks[k - 1], cwA[k], cw_hi)

    heights = ch_hi - ch_lo
    widths = cw_hi - cw_lo

    d_lo_raw = plsc.load_gather(udb, [idx, lane_e])
    d_hi_raw = plsc.load_gather(udb, [idx + 1, lane_e])
    one_f = jnp.full((16,), 1.0, f32)
    d_lo = jnp.where(mks[0], one_f, MIND + _softplus(d_lo_raw))
    d_hi = jnp.where(mks[NBINS - 1], one_f, MIND + _softplus(d_hi_raw))

    delta = heights / widths
    dx = x - ch_lo
    two = d_lo + d_hi - 2.0 * delta
    aq = dx * two + heights * (delta - d_lo)
    bq = heights * d_lo - dx * two
    cq = -delta * dx
    disc = bq * bq - 4.0 * aq * cq
    root = (2.0 * cq) / (-bq - _sqrt_nn(jnp.maximum(disc, 0.0)))
    return root * widths + cw_lo


def _tec_kernel(x_hbm, uw_hbm, uh_hbm, ud_hbm, out_hbm, xb, uwb, uhb, udb, ob, sem):
    c = lax.axis_index("c")
    s = lax.axis_index("s")
    wid = s * 2 + c
    wbase = wid * WELEMS
    lane = lax.iota(jnp.int32, 16)

    def chunk_body(ci, carry):
        ebase = wbase + ci * CHUNK
        copies = [pltpu.make_async_copy(x_hbm.at[pl.ds(ebase, CHUNK)], xb, sem)]
        for k in range(NBINS):
            copies.append(pltpu.make_async_copy(
                uw_hbm.at[pl.ds(k * N + ebase, CHUNK)], uwb.at[k], sem))
            copies.append(pltpu.make_async_copy(
                uh_hbm.at[pl.ds(k * N + ebase, CHUNK)], uhb.at[k], sem))
        for k in range(NBINS + 1):
            copies.append(pltpu.make_async_copy(
                ud_hbm.at[pl.ds(k * N + ebase, CHUNK)], udb.at[k], sem))
        for cp in copies:
            cp.start()
        for cp in copies:
            cp.wait()

        def group_body(g, carry2):
            b = g * 16
            lane_e = b + lane
            uwk = [uwb[k, pl.ds(b, 16)] for k in range(NBINS)]
            uhk = [uhb[k, pl.ds(b, 16)] for k in range(NBINS)]
            x = xb[pl.ds(b, 16)]
            out = _spline_group(x, uwk, uhk, udb, lane_e)
            ob[pl.ds(b, 16)] = out
            return carry2

        lax.fori_loop(0, NGROUPS, group_body, 0)
        pltpu.sync_copy(ob, out_hbm.at[pl.ds(ebase, CHUNK)])
        return carry

    lax.fori_loop(0, NCHUNKS, chunk_body, 0)


@jax.jit
def _run(x, uwt, uht, udt):
    mesh = plsc.VectorSubcoreMesh(core_axis_name="c", subcore_axis_name="s")
    grid_kernel = pl.kernel(
        _tec_kernel,
        out_type=jax.ShapeDtypeStruct((N,), jnp.float32),
        mesh=mesh,
        compiler_params=pltpu.CompilerParams(
            needs_layout_passes=False, use_tc_tiling_on_sc=False),
        scratch_types=[
            pltpu.VMEM((CHUNK,), jnp.float32),
            pltpu.VMEM((NBINS, CHUNK), jnp.float32),
            pltpu.VMEM((NBINS, CHUNK), jnp.float32),
            pltpu.VMEM((NBINS + 1, CHUNK), jnp.float32),
            pltpu.VMEM((CHUNK,), jnp.float32),
            pltpu.SemaphoreType.DMA,
        ],
    )
    return grid_kernel(x, uwt, uht, udt)


def kernel(inputs, unnormalized_widths, unnormalized_heights, unnormalized_derivatives):
    x = inputs.reshape(N)
    uwt = jnp.concatenate([unnormalized_widths[0, 0, :, k] for k in range(NBINS)])
    uht = jnp.concatenate([unnormalized_heights[0, 0, :, k] for k in range(NBINS)])
    udt = jnp.concatenate([unnormalized_derivatives[0, 0, :, k] for k in range(NBINS + 1)])
    out = _run(x, uwt, uht, udt)
    return out.reshape(1, 1, N)
